# baseline (speedup 1.0000x reference)
import jax
import jax.numpy as jnp
from jax import lax
from jax.experimental import pallas as pl
from jax.experimental.pallas import tpu as pltpu
from jax.experimental.pallas import tpu_sc as plsc

_NOISE = 0.001
_SCALE = -1.0 / _NOISE

_NB = 4
_W = 512
_ROWS = 16 * 512
_NPER = _ROWS * _W

_R_SC = 2048
_R_TC = _ROWS - _R_SC
_TC_CHUNK = 1024

_NW = 32
_RPW = _R_SC // _NW
_CROWS = 16
_NCHUNK = _RPW // _CROWS
_L = 16
_U = 8


def _tc_body(x_ref, b_ref, o_ref):
    b = pl.program_id(0)
    t = pl.program_id(1)
    xv = x_ref[...]
    bv = b_ref[...]
    d = jnp.where(jnp.isfinite(bv), bv - xv, 0.0)
    s = _SCALE * jnp.sum(d * d)

    @pl.when(t == 0)
    def _init():
        o_ref[b] = s

    @pl.when(t != 0)
    def _acc():
        o_ref[b] += s


def _chunk_sum(xbuf, bbuf, par, acc):
    ngrp = _CROWS * _W // (_L * _U)
    gper = _W // (_L * _U)

    def vec_body(g, accs):
        r = g // gper
        c0 = (g % gper) * _L * _U
        new = []
        for u in range(_U):
            xv = xbuf[par, r, pl.ds(c0 + u * _L, _L)]
            bv = bbuf[par, r, pl.ds(c0 + u * _L, _L)]
            m = jnp.abs(bv) < jnp.float32(jnp.inf)
            d = jnp.where(m, bv - xv, jnp.float32(0.0))
            new.append(accs[u] + d * d)
        return tuple(new)

    return plsc.parallel_loop(0, ngrp, unroll=2, carry=acc)(vec_body)


def _sc_body(x_hbm, b_hbm, out_hbm, xbuf, bbuf, pbuf, sx0, sx1, sb0, sb1):
    cid = lax.axis_index("c")
    sid = lax.axis_index("s")
    wid = sid * 2 + cid
    r0 = _R_TC + wid * _RPW
    sems = ((sx0, sb0), (sx1, sb1))

    sched = [(b, k) for b in range(_NB) for k in range(_NCHUNK)]
    ntot = len(sched)
    waits = [None, None]

    def issue(ci):
        b, k = sched[ci]
        par = ci % 2
        sx, sb = sems[par]
        row = r0 + k * _CROWS
        hx = pltpu.async_copy(
            x_hbm.at[b, pl.ds(row, _CROWS)], xbuf.at[par], sx)
        hb = pltpu.async_copy(
            b_hbm.at[b, pl.ds(row, _CROWS)], bbuf.at[par], sb)
        waits[par] = (hx, hb)

    issue(0)
    issue(1)
    accs = [tuple(jnp.zeros((_L,), jnp.float32) for _ in range(_U))
            for _ in range(_NB)]
    for ci in range(ntot):
        b, _k = sched[ci]
        par = ci % 2
        hx, hb = waits[par]
        hx.wait()
        hb.wait()
        accs[b] = _chunk_sum(xbuf, bbuf, par, accs[b])
        if ci + 2 < ntot:
            issue(ci + 2)

    for b in range(_NB):
        a = accs[b]
        total = ((a[0] + a[1]) + (a[2] + a[3])) + (
            (a[4] + a[5]) + (a[6] + a[7]))
        pbuf[...] = total
        pltpu.sync_copy(pbuf, out_hbm.at[b, wid])


def kernel(x, batch):
    x2 = x.reshape(_NB, _ROWS, _W)
    b2 = batch.reshape(_NB, _ROWS, _W)

    tc_out = pl.pallas_call(
        _tc_body,
        grid=(_NB, _R_TC // _TC_CHUNK),
        in_specs=[
            pl.BlockSpec((1, _TC_CHUNK, _W), lambda b, t: (b, t, 0)),
            pl.BlockSpec((1, _TC_CHUNK, _W), lambda b, t: (b, t, 0)),
        ],
        out_specs=pl.BlockSpec(
            (_NB,), lambda b, t: (0,), memory_space=pltpu.SMEM
        ),
        out_shape=jax.ShapeDtypeStruct((_NB,), jnp.float32),
    )(x2, b2)

    mesh = plsc.VectorSubcoreMesh(core_axis_name="c", subcore_axis_name="s")
    sc_partial = pl.kernel(
        _sc_body,
        mesh=mesh,
        out_type=jax.ShapeDtypeStruct((_NB, _NW, _L), jnp.float32),
        scratch_types=[
            pltpu.VMEM((2, _CROWS, _W), jnp.float32),
            pltpu.VMEM((2, _CROWS, _W), jnp.float32),
            pltpu.VMEM((_L,), jnp.float32),
            pltpu.SemaphoreType.DMA,
            pltpu.SemaphoreType.DMA,
            pltpu.SemaphoreType.DMA,
            pltpu.SemaphoreType.DMA,
        ],
        compiler_params=pltpu.CompilerParams(use_tc_tiling_on_sc=True),
    )(x2, b2)

    return tc_out + _SCALE * jnp.sum(sc_partial, axis=(1, 2))


# speedup vs baseline: 1.0227x; 1.0227x over previous
import jax
import jax.numpy as jnp
from jax import lax
from jax.experimental import pallas as pl
from jax.experimental.pallas import tpu as pltpu
from jax.experimental.pallas import tpu_sc as plsc

_NOISE = 0.001
_SCALE = -1.0 / _NOISE

_NB = 4
_W = 512
_ROWS = 16 * 512
_NPER = _ROWS * _W

_R_SC = 3072
_R_TC = _ROWS - _R_SC
_TC_CHUNK = 1024

_NW = 32
_RPW = _R_SC // _NW
_CROWS = 32
_NCHUNK = _RPW // _CROWS
_L = 16
_U = 8


def _tc_body(x_ref, b_ref, o_ref):
    b = pl.program_id(0)
    t = pl.program_id(1)
    xv = x_ref[...]
    bv = b_ref[...]
    d = jnp.where(jnp.isfinite(bv), bv - xv, 0.0)
    s = _SCALE * jnp.sum(d * d)

    @pl.when(t == 0)
    def _init():
        o_ref[b] = s

    @pl.when(t != 0)
    def _acc():
        o_ref[b] += s


def _chunk_sum(xbuf, bbuf, par, acc):
    ngrp = _CROWS * _W // (_L * _U)
    gper = _W // (_L * _U)

    def vec_body(g, accs):
        r = g // gper
        c0 = (g % gper) * _L * _U
        new = []
        for u in range(_U):
            xv = xbuf[par, r, pl.ds(c0 + u * _L, _L)]
            bv = bbuf[par, r, pl.ds(c0 + u * _L, _L)]
            m = jnp.abs(bv) < jnp.float32(jnp.inf)
            d = jnp.where(m, bv - xv, jnp.float32(0.0))
            new.append(accs[u] + d * d)
        return tuple(new)

    return plsc.parallel_loop(0, ngrp, unroll=2, carry=acc)(vec_body)


def _sc_body(x_hbm, b_hbm, out_hbm, xbuf, bbuf, pbuf, sx0, sx1, sb0, sb1):
    cid = lax.axis_index("c")
    sid = lax.axis_index("s")
    wid = sid * 2 + cid
    r0 = _R_TC + wid * _RPW
    sems = ((sx0, sb0), (sx1, sb1))

    sched = [(b, k) for b in range(_NB) for k in range(_NCHUNK)]
    ntot = len(sched)
    waits = [None, None]

    def issue(ci):
        b, k = sched[ci]
        par = ci % 2
        sx, sb = sems[par]
        row = r0 + k * _CROWS
        hx = pltpu.async_copy(
            x_hbm.at[b, pl.ds(row, _CROWS)], xbuf.at[par], sx)
        hb = pltpu.async_copy(
            b_hbm.at[b, pl.ds(row, _CROWS)], bbuf.at[par], sb)
        waits[par] = (hx, hb)

    issue(0)
    issue(1)
    accs = [tuple(jnp.zeros((_L,), jnp.float32) for _ in range(_U))
            for _ in range(_NB)]
    for ci in range(ntot):
        b, _k = sched[ci]
        par = ci % 2
        hx, hb = waits[par]
        hx.wait()
        hb.wait()
        accs[b] = _chunk_sum(xbuf, bbuf, par, accs[b])
        if ci + 2 < ntot:
            issue(ci + 2)

    for b in range(_NB):
        a = accs[b]
        total = ((a[0] + a[1]) + (a[2] + a[3])) + (
            (a[4] + a[5]) + (a[6] + a[7]))
        pbuf[...] = total
        pltpu.sync_copy(pbuf, out_hbm.at[b, wid])


def kernel(x, batch):
    x2 = x.reshape(_NB, _ROWS, _W)
    b2 = batch.reshape(_NB, _ROWS, _W)

    tc_out = pl.pallas_call(
        _tc_body,
        grid=(_NB, _R_TC // _TC_CHUNK),
        in_specs=[
            pl.BlockSpec((1, _TC_CHUNK, _W), lambda b, t: (b, t, 0)),
            pl.BlockSpec((1, _TC_CHUNK, _W), lambda b, t: (b, t, 0)),
        ],
        out_specs=pl.BlockSpec(
            (_NB,), lambda b, t: (0,), memory_space=pltpu.SMEM
        ),
        out_shape=jax.ShapeDtypeStruct((_NB,), jnp.float32),
    )(x2, b2)

    mesh = plsc.VectorSubcoreMesh(core_axis_name="c", subcore_axis_name="s")
    sc_partial = pl.kernel(
        _sc_body,
        mesh=mesh,
        out_type=jax.ShapeDtypeStruct((_NB, _NW, _L), jnp.float32),
        scratch_types=[
            pltpu.VMEM((2, _CROWS, _W), jnp.float32),
            pltpu.VMEM((2, _CROWS, _W), jnp.float32),
            pltpu.VMEM((_L,), jnp.float32),
            pltpu.SemaphoreType.DMA,
            pltpu.SemaphoreType.DMA,
            pltpu.SemaphoreType.DMA,
            pltpu.SemaphoreType.DMA,
        ],
        compiler_params=pltpu.CompilerParams(use_tc_tiling_on_sc=True),
    )(x2, b2)

    return tc_out + _SCALE * jnp.sum(sc_partial, axis=(1, 2))
